# baseline (speedup 1.0000x reference)
import functools

import jax
import jax.numpy as jnp
from jax import lax
from jax.experimental import pallas as pl
from jax.experimental.pallas import tpu as pltpu
from jax.experimental.pallas import tpu_sc as plsc

_B = 4
_S = 8192
_D = 1024
_SPLIT = 5120

_info = plsc.get_sparse_core_info()
_NC = _info.num_cores
_NS = _info.num_subcores
_NW = _NC * _NS
_ROWS_PER_W = _SPLIT // _NW
_R = 4
_JOBS = _ROWS_PER_W // _R

_mesh = plsc.VectorSubcoreMesh(core_axis_name="c", subcore_axis_name="s")


@functools.partial(
    pl.kernel,
    out_type=jax.ShapeDtypeStruct((_B, _S, _D), jnp.float32),
    mesh=_mesh,
    scratch_types=[
        pltpu.VMEM((_R, _D), jnp.float32),
        pltpu.VMEM((_R, _D), jnp.float32),
        pltpu.VMEM((_B * _R, _D), jnp.float32),
        pltpu.VMEM((_B * _R, _D), jnp.float32),
        pltpu.VMEM((_B * _R, _D), jnp.float32),
        pltpu.VMEM((_B * _R, _D), jnp.float32),
        pltpu.SemaphoreType.DMA,
        pltpu.SemaphoreType.DMA,
        pltpu.SemaphoreType.DMA,
        pltpu.SemaphoreType.DMA,
    ],
)
def _pos_add_sc(x_hbm, pos_hbm, out_hbm, pos_v0, pos_v1, x_v0, x_v1,
                o_v0, o_v1, sl0, sl1, ss0, ss1):
    wid = lax.axis_index("s") * _NC + lax.axis_index("c")
    base = wid * _ROWS_PER_W
    pos_v = (pos_v0, pos_v1)
    x_v = (x_v0, x_v1)
    o_v = (o_v0, o_v1)
    sem_l = (sl0, sl1)
    sem_s = (ss0, ss1)

    def issue_loads(k, c):
        row0 = base + k * _R
        pltpu.async_copy(pos_hbm.at[pl.ds(row0, _R), :], pos_v[c], sem_l[c])
        for b in range(_B):
            pltpu.async_copy(
                x_hbm.at[b, pl.ds(row0, _R), :],
                x_v[c].at[pl.ds(b * _R, _R), :],
                sem_l[c],
            )

    def wait_loads(c):
        pltpu.make_async_copy(
            pos_hbm.at[pl.ds(0, _R), :], pos_v[c], sem_l[c]).wait()
        pltpu.make_async_copy(
            x_hbm.at[0, pl.ds(0, _B * _R), :], x_v[c], sem_l[c]).wait()

    def issue_stores(k, c):
        row0 = base + k * _R
        for b in range(_B):
            pltpu.async_copy(
                o_v[c].at[pl.ds(b * _R, _R), :],
                out_hbm.at[b, pl.ds(row0, _R), :],
                sem_s[c],
            )

    def wait_stores(c):
        pltpu.make_async_copy(
            o_v[c], out_hbm.at[0, pl.ds(0, _B * _R), :], sem_s[c]).wait()

    def compute(c):
        xc = x_v[c]
        oc = o_v[c]
        pc = pos_v[c]

        @plsc.parallel_loop(0, _D, step=16, unroll=4)
        def add_body(i):
            for r in range(_R):
                p = pc[r, pl.ds(i, 16)]
                for b in range(_B):
                    row = b * _R + r
                    oc[row, pl.ds(i, 16)] = xc[row, pl.ds(i, 16)] + p

    issue_loads(0, 0)
    issue_loads(1, 1)

    def iter_body(k0, _):
        for c in (0, 1):
            k = k0 * 2 + c
            wait_loads(c)
            @pl.when(k0 >= 1)
            def _():
                wait_stores(c)
            compute(c)
            @pl.when(k0 <= _JOBS // 2 - 2)
            def _():
                issue_loads(k + 2, c)
            issue_stores(k, c)
        return 0

    lax.fori_loop(0, _JOBS // 2, iter_body, 0)
    wait_stores(0)
    wait_stores(1)


_TC_ROWS = _S - _SPLIT
_TC_BS = 256


def _pos_add_tc_body(x_ref, pos_ref, out_ref):
    out_ref[...] = x_ref[...] + pos_ref[...][None, :, :]


_TC_OFF = _SPLIT // _TC_BS

_pos_add_tc = pl.pallas_call(
    _pos_add_tc_body,
    grid=(_TC_ROWS // _TC_BS,),
    in_specs=[
        pl.BlockSpec((_B, _TC_BS, _D), lambda i: (0, i + _TC_OFF, 0)),
        pl.BlockSpec((_TC_BS, _D), lambda i: (i + _TC_OFF, 0)),
    ],
    out_specs=pl.BlockSpec((_B, _TC_BS, _D), lambda i: (0, i, 0)),
    out_shape=jax.ShapeDtypeStruct((_B, _TC_ROWS, _D), jnp.float32),
)


def kernel(x, position_matrix):
    out_sc = _pos_add_sc(x, position_matrix)
    out_tc = _pos_add_tc(x, position_matrix)
    return lax.dynamic_update_slice(out_sc, out_tc, (0, _SPLIT, 0))


# speedup vs baseline: 1.2150x; 1.2150x over previous
import functools

import jax
import jax.numpy as jnp
from jax import lax
from jax.experimental import pallas as pl
from jax.experimental.pallas import tpu as pltpu
from jax.experimental.pallas import tpu_sc as plsc

_B = 4
_S = 8192
_D = 1024
_SPLIT = 4096

_info = plsc.get_sparse_core_info()
_NC = _info.num_cores
_NS = _info.num_subcores
_NW = _NC * _NS
_ROWS_PER_W = _SPLIT // _NW
_R = 4
_JOBS = _ROWS_PER_W // _R

_mesh = plsc.VectorSubcoreMesh(core_axis_name="c", subcore_axis_name="s")


@functools.partial(
    pl.kernel,
    out_type=jax.ShapeDtypeStruct((_B, _S, _D), jnp.float32),
    mesh=_mesh,
    scratch_types=[
        pltpu.VMEM((_R, _D), jnp.float32),
        pltpu.VMEM((_R, _D), jnp.float32),
        pltpu.VMEM((_B * _R, _D), jnp.float32),
        pltpu.VMEM((_B * _R, _D), jnp.float32),
        pltpu.VMEM((_B * _R, _D), jnp.float32),
        pltpu.VMEM((_B * _R, _D), jnp.float32),
        pltpu.SemaphoreType.DMA,
        pltpu.SemaphoreType.DMA,
        pltpu.SemaphoreType.DMA,
        pltpu.SemaphoreType.DMA,
    ],
)
def _pos_add_sc(x_hbm, pos_hbm, out_hbm, pos_v0, pos_v1, x_v0, x_v1,
                o_v0, o_v1, sl0, sl1, ss0, ss1):
    wid = lax.axis_index("s") * _NC + lax.axis_index("c")
    base = wid * _ROWS_PER_W
    pos_v = (pos_v0, pos_v1)
    x_v = (x_v0, x_v1)
    o_v = (o_v0, o_v1)
    sem_l = (sl0, sl1)
    sem_s = (ss0, ss1)

    def issue_loads(k, c):
        row0 = base + k * _R
        pltpu.async_copy(pos_hbm.at[pl.ds(row0, _R), :], pos_v[c], sem_l[c])
        for b in range(_B):
            pltpu.async_copy(
                x_hbm.at[b, pl.ds(row0, _R), :],
                x_v[c].at[pl.ds(b * _R, _R), :],
                sem_l[c],
            )

    def wait_loads(c):
        pltpu.make_async_copy(
            pos_hbm.at[pl.ds(0, _R), :], pos_v[c], sem_l[c]).wait()
        pltpu.make_async_copy(
            x_hbm.at[0, pl.ds(0, _B * _R), :], x_v[c], sem_l[c]).wait()

    def issue_stores(k, c):
        row0 = base + k * _R
        for b in range(_B):
            pltpu.async_copy(
                o_v[c].at[pl.ds(b * _R, _R), :],
                out_hbm.at[b, pl.ds(row0, _R), :],
                sem_s[c],
            )

    def wait_stores(c):
        pltpu.make_async_copy(
            o_v[c], out_hbm.at[0, pl.ds(0, _B * _R), :], sem_s[c]).wait()

    def compute(c):
        xc = x_v[c]
        oc = o_v[c]
        pc = pos_v[c]

        @plsc.parallel_loop(0, _D, step=16, unroll=4)
        def add_body(i):
            for r in range(_R):
                p = pc[r, pl.ds(i, 16)]
                for b in range(_B):
                    row = b * _R + r
                    oc[row, pl.ds(i, 16)] = xc[row, pl.ds(i, 16)] + p

    issue_loads(0, 0)
    issue_loads(1, 1)

    def iter_body(k0, _):
        for c in (0, 1):
            k = k0 * 2 + c
            wait_loads(c)
            @pl.when(k0 >= 1)
            def _():
                wait_stores(c)
            compute(c)
            @pl.when(k0 <= _JOBS // 2 - 2)
            def _():
                issue_loads(k + 2, c)
            issue_stores(k, c)
        return 0

    lax.fori_loop(0, _JOBS // 2, iter_body, 0)
    wait_stores(0)
    wait_stores(1)


_TC_ROWS = _S - _SPLIT
_TC_BS = 256


def _pos_add_tc_body(acc_ref, x_ref, pos_ref, out_ref):
    del acc_ref
    out_ref[...] = x_ref[...] + pos_ref[...][None, :, :]


_TC_OFF = _SPLIT // _TC_BS

_pos_add_tc = pl.pallas_call(
    _pos_add_tc_body,
    grid=(_TC_ROWS // _TC_BS,),
    in_specs=[
        pl.BlockSpec(memory_space=pl.ANY),
        pl.BlockSpec((_B, _TC_BS, _D), lambda i: (0, i + _TC_OFF, 0)),
        pl.BlockSpec((_TC_BS, _D), lambda i: (i + _TC_OFF, 0)),
    ],
    out_specs=pl.BlockSpec((_B, _TC_BS, _D), lambda i: (0, i + _TC_OFF, 0)),
    out_shape=jax.ShapeDtypeStruct((_B, _S, _D), jnp.float32),
    input_output_aliases={0: 0},
)


def kernel(x, position_matrix):
    out_sc = _pos_add_sc(x, position_matrix)
    return _pos_add_tc(out_sc, x, position_matrix)


# speedup vs baseline: 1.2162x; 1.0009x over previous
import functools

import jax
import jax.numpy as jnp
from jax import lax
from jax.experimental import pallas as pl
from jax.experimental.pallas import tpu as pltpu
from jax.experimental.pallas import tpu_sc as plsc

_B = 4
_S = 8192
_D = 1024
_SPLIT = 4096

_info = plsc.get_sparse_core_info()
_NC = _info.num_cores
_NS = _info.num_subcores
_NW = _NC * _NS
_ROWS_PER_W = _SPLIT // _NW
_R = 4
_JOBS = _ROWS_PER_W // _R

_mesh = plsc.VectorSubcoreMesh(core_axis_name="c", subcore_axis_name="s")


@functools.partial(
    pl.kernel,
    out_type=jax.ShapeDtypeStruct((_B, _S, _D), jnp.float32),
    mesh=_mesh,
    scratch_types=[
        pltpu.VMEM((_R, _D), jnp.float32),
        pltpu.VMEM((_R, _D), jnp.float32),
        pltpu.VMEM((_B * _R, _D), jnp.float32),
        pltpu.VMEM((_B * _R, _D), jnp.float32),
        pltpu.VMEM((_B * _R, _D), jnp.float32),
        pltpu.VMEM((_B * _R, _D), jnp.float32),
        pltpu.SemaphoreType.DMA,
        pltpu.SemaphoreType.DMA,
        pltpu.SemaphoreType.DMA,
        pltpu.SemaphoreType.DMA,
    ],
)
def _pos_add_sc(x_hbm, pos_hbm, out_hbm, pos_v0, pos_v1, x_v0, x_v1,
                o_v0, o_v1, sl0, sl1, ss0, ss1):
    wid = lax.axis_index("s") * _NC + lax.axis_index("c")
    base = wid * _ROWS_PER_W
    pos_v = (pos_v0, pos_v1)
    x_v = (x_v0, x_v1)
    o_v = (o_v0, o_v1)
    sem_l = (sl0, sl1)
    sem_s = (ss0, ss1)

    def issue_loads(k, c):
        row0 = base + k * _R
        pltpu.async_copy(pos_hbm.at[pl.ds(row0, _R), :], pos_v[c], sem_l[c])
        for b in range(_B):
            pltpu.async_copy(
                x_hbm.at[b, pl.ds(row0, _R), :],
                x_v[c].at[pl.ds(b * _R, _R), :],
                sem_l[c],
            )

    def wait_loads(c):
        pltpu.make_async_copy(
            pos_hbm.at[pl.ds(0, _R), :], pos_v[c], sem_l[c]).wait()
        pltpu.make_async_copy(
            x_hbm.at[0, pl.ds(0, _B * _R), :], x_v[c], sem_l[c]).wait()

    def issue_stores(k, c):
        row0 = base + k * _R
        for b in range(_B):
            pltpu.async_copy(
                o_v[c].at[pl.ds(b * _R, _R), :],
                out_hbm.at[b, pl.ds(row0, _R), :],
                sem_s[c],
            )

    def wait_stores(c):
        pltpu.make_async_copy(
            o_v[c], out_hbm.at[0, pl.ds(0, _B * _R), :], sem_s[c]).wait()

    def compute(c):
        xc = x_v[c]
        oc = o_v[c]
        pc = pos_v[c]

        @plsc.parallel_loop(0, _D, step=16, unroll=4)
        def add_body(i):
            for r in range(_R):
                p = pc[r, pl.ds(i, 16)]
                for b in range(_B):
                    row = b * _R + r
                    oc[row, pl.ds(i, 16)] = xc[row, pl.ds(i, 16)] + p

    issue_loads(0, 0)
    issue_loads(1, 1)

    def iter_body(k0, _):
        for c in (0, 1):
            k = k0 * 2 + c
            wait_loads(c)
            @pl.when(k0 >= 1)
            def _():
                wait_stores(c)
            compute(c)
            @pl.when(k0 <= _JOBS // 2 - 2)
            def _():
                issue_loads(k + 2, c)
            issue_stores(k, c)
        return 0

    lax.fori_loop(0, _JOBS // 2, iter_body, 0)
    wait_stores(0)
    wait_stores(1)


_TC_ROWS = _S - _SPLIT
_TC_BS = 512


def _pos_add_tc_body(acc_ref, x_ref, pos_ref, out_ref):
    del acc_ref
    out_ref[...] = x_ref[...] + pos_ref[...][None, :, :]


_TC_OFF = _SPLIT // _TC_BS

_pos_add_tc = pl.pallas_call(
    _pos_add_tc_body,
    grid=(_TC_ROWS // _TC_BS,),
    in_specs=[
        pl.BlockSpec(memory_space=pl.ANY),
        pl.BlockSpec((_B, _TC_BS, _D), lambda i: (0, i + _TC_OFF, 0)),
        pl.BlockSpec((_TC_BS, _D), lambda i: (i + _TC_OFF, 0)),
    ],
    out_specs=pl.BlockSpec((_B, _TC_BS, _D), lambda i: (0, i + _TC_OFF, 0)),
    out_shape=jax.ShapeDtypeStruct((_B, _S, _D), jnp.float32),
    input_output_aliases={0: 0},
)


def kernel(x, position_matrix):
    out_sc = _pos_add_sc(x, position_matrix)
    return _pos_add_tc(out_sc, x, position_matrix)


# speedup vs baseline: 1.2248x; 1.0071x over previous
import functools

import jax
import jax.numpy as jnp
from jax import lax
from jax.experimental import pallas as pl
from jax.experimental.pallas import tpu as pltpu
from jax.experimental.pallas import tpu_sc as plsc

_B = 4
_S = 8192
_D = 1024
_SPLIT = 3584

_info = plsc.get_sparse_core_info()
_NC = _info.num_cores
_NS = _info.num_subcores
_NW = _NC * _NS
_ROWS_PER_W = _SPLIT // _NW
_R = 4
_JOBS = _ROWS_PER_W // _R

_mesh = plsc.VectorSubcoreMesh(core_axis_name="c", subcore_axis_name="s")


@functools.partial(
    pl.kernel,
    out_type=jax.ShapeDtypeStruct((_B, _S, _D), jnp.float32),
    mesh=_mesh,
    scratch_types=[
        pltpu.VMEM((_R, _D), jnp.float32),
        pltpu.VMEM((_R, _D), jnp.float32),
        pltpu.VMEM((_B * _R, _D), jnp.float32),
        pltpu.VMEM((_B * _R, _D), jnp.float32),
        pltpu.VMEM((_B * _R, _D), jnp.float32),
        pltpu.VMEM((_B * _R, _D), jnp.float32),
        pltpu.SemaphoreType.DMA,
        pltpu.SemaphoreType.DMA,
        pltpu.SemaphoreType.DMA,
        pltpu.SemaphoreType.DMA,
    ],
)
def _pos_add_sc(x_hbm, pos_hbm, out_hbm, pos_v0, pos_v1, x_v0, x_v1,
                o_v0, o_v1, sl0, sl1, ss0, ss1):
    wid = lax.axis_index("s") * _NC + lax.axis_index("c")
    base = wid * _ROWS_PER_W
    pos_v = (pos_v0, pos_v1)
    x_v = (x_v0, x_v1)
    o_v = (o_v0, o_v1)
    sem_l = (sl0, sl1)
    sem_s = (ss0, ss1)

    def issue_loads(k, c):
        row0 = base + k * _R
        pltpu.async_copy(pos_hbm.at[pl.ds(row0, _R), :], pos_v[c], sem_l[c])
        for b in range(_B):
            pltpu.async_copy(
                x_hbm.at[b, pl.ds(row0, _R), :],
                x_v[c].at[pl.ds(b * _R, _R), :],
                sem_l[c],
            )

    def wait_loads(c):
        pltpu.make_async_copy(
            pos_hbm.at[pl.ds(0, _R), :], pos_v[c], sem_l[c]).wait()
        pltpu.make_async_copy(
            x_hbm.at[0, pl.ds(0, _B * _R), :], x_v[c], sem_l[c]).wait()

    def issue_stores(k, c):
        row0 = base + k * _R
        for b in range(_B):
            pltpu.async_copy(
                o_v[c].at[pl.ds(b * _R, _R), :],
                out_hbm.at[b, pl.ds(row0, _R), :],
                sem_s[c],
            )

    def wait_stores(c):
        pltpu.make_async_copy(
            o_v[c], out_hbm.at[0, pl.ds(0, _B * _R), :], sem_s[c]).wait()

    def compute(c):
        xc = x_v[c]
        oc = o_v[c]
        pc = pos_v[c]

        @plsc.parallel_loop(0, _D, step=16, unroll=4)
        def add_body(i):
            for r in range(_R):
                p = pc[r, pl.ds(i, 16)]
                for b in range(_B):
                    row = b * _R + r
                    oc[row, pl.ds(i, 16)] = xc[row, pl.ds(i, 16)] + p

    issue_loads(0, 0)
    issue_loads(1, 1)

    def iter_body(k0, _):
        for c in (0, 1):
            k = k0 * 2 + c
            wait_loads(c)
            @pl.when(k0 >= 1)
            def _():
                wait_stores(c)
            compute(c)
            @pl.when(k0 <= _JOBS // 2 - 2)
            def _():
                issue_loads(k + 2, c)
            issue_stores(k, c)
        return 0

    lax.fori_loop(0, _JOBS // 2, iter_body, 0)
    wait_stores(0)
    wait_stores(1)


_TC_ROWS = _S - _SPLIT
_TC_BS = 512


def _pos_add_tc_body(acc_ref, x_ref, pos_ref, out_ref):
    del acc_ref
    out_ref[...] = x_ref[...] + pos_ref[...][None, :, :]


_TC_OFF = _SPLIT // _TC_BS

_pos_add_tc = pl.pallas_call(
    _pos_add_tc_body,
    grid=(_TC_ROWS // _TC_BS,),
    in_specs=[
        pl.BlockSpec(memory_space=pl.ANY),
        pl.BlockSpec((_B, _TC_BS, _D), lambda i: (0, i + _TC_OFF, 0)),
        pl.BlockSpec((_TC_BS, _D), lambda i: (i + _TC_OFF, 0)),
    ],
    out_specs=pl.BlockSpec((_B, _TC_BS, _D), lambda i: (0, i + _TC_OFF, 0)),
    out_shape=jax.ShapeDtypeStruct((_B, _S, _D), jnp.float32),
    input_output_aliases={0: 0},
)


def kernel(x, position_matrix):
    out_sc = _pos_add_sc(x, position_matrix)
    return _pos_add_tc(out_sc, x, position_matrix)
